# Initial kernel scaffold; baseline (speedup 1.0000x reference)
import jax
import jax.numpy as jnp
from jax.experimental import pallas as pl


def kernel(sub_idx, rel_idx, obj_idx, node_table, rel_table):
    raise NotImplementedError("write your pallas kernel here")


# speedup vs baseline: 2.9839x; 2.9839x over previous
import functools

import jax
import jax.numpy as jnp
from jax import lax
from jax.experimental import pallas as pl
from jax.experimental.pallas import tpu as pltpu
from jax.experimental.pallas import tpu_sc as plsc

NUM_ENTITY = 100000
NUM_REL = 500
EMBED_DIM = 128
BATCH = 16384

_INFO = plsc.get_sparse_core_info()
_NC, _NS, _L = _INFO.num_cores, _INFO.num_subcores, _INFO.num_lanes
_NW = _NC * _NS
_BPW = BATCH // _NW


def _sc_kernel(sub_hbm, rel_hbm, obj_hbm, node_hbm, rel_t_hbm, out_hbm,
               sub_v, rel_v, obj_v, rows_v, sem):
    wid = lax.axis_index("s") * _NC + lax.axis_index("c")
    base = wid * _BPW

    idx_bufs = (sub_v, rel_v, obj_v)
    for src, dst in zip((sub_hbm, rel_hbm, obj_hbm), idx_bufs):
        pltpu.sync_copy(src.at[pl.ds(base, _BPW)], dst)

    for buf in idx_bufs:
        for i in range(_BPW // _L):
            sl = pl.ds(i * _L, _L)
            buf[sl] = buf[sl] + 1

    tables = (node_hbm, rel_t_hbm, node_hbm)
    for t in range(3):
        pltpu.async_copy(tables[t].at[idx_bufs[t]], rows_v, sem).wait()
        pltpu.sync_copy(rows_v, out_hbm.at[t, pl.ds(base, _BPW)])


@jax.jit
def _run(sub_idx, rel_idx, obj_idx, node_table, rel_table):
    k = functools.partial(
        pl.kernel,
        mesh=plsc.VectorSubcoreMesh(core_axis_name="c", subcore_axis_name="s"),
        out_type=jax.ShapeDtypeStruct((3, BATCH, EMBED_DIM), jnp.float32),
        scratch_types=[
            pltpu.VMEM((_BPW,), jnp.int32),
            pltpu.VMEM((_BPW,), jnp.int32),
            pltpu.VMEM((_BPW,), jnp.int32),
            pltpu.VMEM((_BPW, EMBED_DIM), jnp.float32),
            pltpu.SemaphoreType.DMA,
        ],
    )(_sc_kernel)
    return k(sub_idx, rel_idx, obj_idx, node_table, rel_table)


def kernel(sub_idx, rel_idx, obj_idx, node_table, rel_table):
    return _run(sub_idx.astype(jnp.int32), rel_idx.astype(jnp.int32),
                obj_idx.astype(jnp.int32), node_table, rel_table)
